# baseline (speedup 1.0000x reference)
import jax
import jax.numpy as jnp
from jax import lax
from jax.experimental import pallas as pl
from jax.experimental.pallas import tpu as pltpu
from jax.experimental.pallas import tpu_sc as plsc

_D = 64
_DP = 128
_V = 1000000
_NPAIR = _V // 2
_NB = 4096
_NT = 200
_NC, _NS, _L = 2, 16, 16
_NW = _NC * _NS
_BPW = _NB // _NW
_TPAD = 133
_SCALE = 8.0

_W = 128
_NCH = _V // _W
_VTAIL = _NCH * _W


def _fmt_body(tabt_hbm, tail_hbm, tp_hbm, in0, in1, out0, out1,
              isem0, isem1, osem0, osem1):
    ins = (in0, in1)
    outs = (out0, out1)
    isems = (isem0, isem1)
    osems = (osem0, osem1)
    wid = lax.axis_index("s") * _NC + lax.axis_index("c")
    lane = lax.iota(jnp.int32, _L)

    def start_read(i, b):
        pltpu.async_copy(tabt_hbm.at[:, pl.ds(i * _W, _W)], ins[b], isems[b])

    def wait_read(i, b):
        pltpu.make_async_copy(tabt_hbm.at[:, pl.ds(i * _W, _W)], ins[b],
                              isems[b]).wait()

    def start_write(i, b):
        pltpu.async_copy(outs[b].at[:, pl.ds(0, _DP)],
                         tp_hbm.at[pl.ds(i * (_W // 2), _W // 2)], osems[b])

    def wait_write(i, b):
        pltpu.make_async_copy(outs[b].at[:, pl.ds(0, _DP)],
                              tp_hbm.at[pl.ds(i * (_W // 2), _W // 2)],
                              osems[b]).wait()

    def transpose(b, nl):
        rvecs = []
        cadds = []
        for lg in range(nl // _L):
            l = lane + lg * _L
            rvecs.append(lax.shift_right_logical(l, 1))
            cadds.append((l & 1) * _D)

        @plsc.parallel_loop(0, _D, unroll=8)
        def _(c):
            for lg in range(nl // _L):
                vals = ins[b][c, pl.ds(lg * _L, _L)]
                plsc.store_scatter(outs[b], [rvecs[lg], cadds[lg] + c], vals)

    nloc = _NCH // _NW + 1
    start_read(wid, 0)

    def chunk_body(g, carry):
        for b in range(2):
            gg = 2 * g + b
            i = wid + gg * _NW

            @pl.when(i < _NCH)
            def _():
                wait_read(i, b)

                @pl.when(i + _NW < _NCH)
                def _():
                    start_read(i + _NW, 1 - b)

                @pl.when(gg >= 2)
                def _():
                    wait_write(i - 2 * _NW, b)

                transpose(b, _W)
                start_write(i, b)
        return carry

    lax.fori_loop(0, (nloc + 1) // 2, chunk_body, 0)
    last = wid + (_NCH // _NW - 1) * _NW
    wait_write(last, (_NCH // _NW - 1) % 2)
    wait_write(last - _NW, (_NCH // _NW) % 2)

    @pl.when(wid == _NW - 1)
    def _():
        pltpu.sync_copy(tail_hbm, ins[0])
        transpose(0, _D)
        pltpu.sync_copy(outs[0].at[pl.ds(0, _D // 2), pl.ds(0, _DP)],
                        tp_hbm.at[pl.ds(_VTAIL // 2, _D // 2)])


def _emb_body(idx_hbm, tablep_hbm, out_hbm, idx_v, lvecs_v, pidx_v, po_v,
              rows0, rows1, rows2, rows3, tr0, tr1,
              gsem0, gsem1, gsem2, gsem3, osem0, osem1):
    rows = (rows0, rows1, rows2, rows3)
    trs = (tr0, tr1)
    gsems = (gsem0, gsem1, gsem2, gsem3)
    osems = (osem0, osem1)
    wid = lax.axis_index("s") * _NC + lax.axis_index("c")
    b0 = wid * _BPW

    lane = lax.iota(jnp.int32, _L)

    def fill_body(i, v):
        lvecs_v[i, :] = v
        return v + 1

    lax.fori_loop(0, _BPW, fill_body, lane * 0)

    def prep_pidx(tl, b):
        for g in range(_BPW // _L):
            sl = pl.ds(g * _L, _L)
            v = idx_v[tl, sl]
            pidx_v[b, sl] = lax.shift_right_logical(v, 1)
            po_v[b, sl] = (v & 1) * _D

    def start_gather(tl, b):
        pltpu.async_copy(tablep_hbm.at[pidx_v.at[b]], rows[b], gsems[b])

    def wait_gather(tl, b):
        pltpu.make_async_copy(tablep_hbm.at[pidx_v.at[b]], rows[b],
                              gsems[b]).wait()

    def start_write(t, b):
        pltpu.async_copy(trs[b].at[:, pl.ds(0, _BPW)],
                         out_hbm.at[t, :, pl.ds(b0, _BPW)], osems[b])

    def wait_write(t, b):
        pltpu.make_async_copy(trs[b].at[:, pl.ds(0, _BPW)],
                              out_hbm.at[t, :, pl.ds(b0, _BPW)],
                              osems[b]).wait()

    def transpose_scale(rb, wb):
        cids = [lane + gc * _L for gc in range(_D // _L)]

        @plsc.parallel_loop(0, _BPW, unroll=8)
        def _(l):
            lvec = lvecs_v[l, :]
            po = po_v[rb, pl.ds(l, _L)][0]
            for gc in range(_D // _L):
                vals = rows[rb][l, pl.ds(po + gc * _L, _L)]
                plsc.store_scatter(trs[wb], [cids[gc], lvec], vals * _SCALE)

    for t0, ht in ((0, 96), (96, 104)):
        pltpu.sync_copy(idx_hbm.at[pl.ds(t0, ht), pl.ds(b0, _BPW)],
                        idx_v.at[pl.ds(0, ht)])
        for tl in range(3):
            prep_pidx(tl, tl)
            start_gather(tl, tl)

        def quad_body(g, carry):
            for b in range(4):
                tl = 4 * g + b
                t = t0 + tl
                wb = b % 2
                wait_gather(tl, b)

                @pl.when(tl + 3 < ht)
                def _():
                    prep_pidx(tl + 3, (b + 3) % 4)
                    start_gather(tl + 3, (b + 3) % 4)

                @pl.when(tl >= 2)
                def _():
                    wait_write(t - 2, wb)

                transpose_scale(b, wb)
                start_write(t, wb)
            return carry

        lax.fori_loop(0, ht // 4, quad_body, 0)
        wait_write(t0 + ht - 2, 0)
        wait_write(t0 + ht - 1, 1)


def _mesh():
    return plsc.VectorSubcoreMesh(
        core_axis_name="c", subcore_axis_name="s",
        num_cores=_NC, num_subcores=_NS,
    )


@jax.jit
def _run(idx_tb, tab_t, tail):
    fmt = pl.kernel(
        _fmt_body,
        out_type=jax.ShapeDtypeStruct((_NPAIR, _DP), jnp.float32),
        mesh=_mesh(),
        scratch_types=[
            pltpu.VMEM((_D, _W), jnp.float32),
            pltpu.VMEM((_D, _W), jnp.float32),
            pltpu.VMEM((_W // 2, _TPAD), jnp.float32),
            pltpu.VMEM((_W // 2, _TPAD), jnp.float32),
            pltpu.SemaphoreType.DMA,
            pltpu.SemaphoreType.DMA,
            pltpu.SemaphoreType.DMA,
            pltpu.SemaphoreType.DMA,
        ],
        compiler_params=pltpu.CompilerParams(
            use_tc_tiling_on_sc=True, needs_layout_passes=False),
    )
    tablep = fmt(tab_t, tail)

    emb = pl.kernel(
        _emb_body,
        out_type=jax.ShapeDtypeStruct((_NT, _D, _NB), jnp.float32),
        mesh=_mesh(),
        scratch_types=[
            pltpu.VMEM((104, _BPW), jnp.int32),
            pltpu.VMEM((_BPW, _L), jnp.int32),
            pltpu.VMEM((4, _BPW), jnp.int32),
            pltpu.VMEM((4, _BPW + _L), jnp.int32),
            pltpu.VMEM((_BPW, _DP), jnp.float32),
            pltpu.VMEM((_BPW, _DP), jnp.float32),
            pltpu.VMEM((_BPW, _DP), jnp.float32),
            pltpu.VMEM((_BPW, _DP), jnp.float32),
            pltpu.VMEM((_D, _TPAD), jnp.float32),
            pltpu.VMEM((_D, _TPAD), jnp.float32),
            pltpu.SemaphoreType.DMA,
            pltpu.SemaphoreType.DMA,
            pltpu.SemaphoreType.DMA,
            pltpu.SemaphoreType.DMA,
            pltpu.SemaphoreType.DMA,
            pltpu.SemaphoreType.DMA,
        ],
        compiler_params=pltpu.CompilerParams(
            use_tc_tiling_on_sc=True, needs_layout_passes=False),
    )
    return emb(idx_tb, tablep)


def kernel(inputs, table):
    idx_tb = inputs.T
    tab_t = table.T
    tail = jnp.pad(lax.slice(tab_t, (0, _VTAIL), (_D, _V)),
                   ((0, 0), (0, _W - (_V - _VTAIL))))
    out_tcb = _run(idx_tb, tab_t, tail)
    return out_tcb.transpose(2, 0, 1)


# speedup vs baseline: 1.0017x; 1.0017x over previous
import jax
import jax.numpy as jnp
from jax import lax
from jax.experimental import pallas as pl
from jax.experimental.pallas import tpu as pltpu
from jax.experimental.pallas import tpu_sc as plsc

_D = 64
_DP = 128
_V = 1000000
_NPAIR = _V // 2
_NB = 4096
_NT = 200
_NC, _NS, _L = 2, 16, 16
_NW = _NC * _NS
_BPW = _NB // _NW
_TPAD = 133
_SCALE = 8.0

_W = 256
_NCH = _V // _W
_VTAIL = _NCH * _W


def _fmt_body(tabt_hbm, tail_hbm, tp_hbm, in0, in1, out0, out1,
              isem0, isem1, osem0, osem1):
    ins = (in0, in1)
    outs = (out0, out1)
    isems = (isem0, isem1)
    osems = (osem0, osem1)
    wid = lax.axis_index("s") * _NC + lax.axis_index("c")
    lane = lax.iota(jnp.int32, _L)

    def start_read(i, b):
        pltpu.async_copy(tabt_hbm.at[:, pl.ds(i * _W, _W)], ins[b], isems[b])

    def wait_read(i, b):
        pltpu.make_async_copy(tabt_hbm.at[:, pl.ds(i * _W, _W)], ins[b],
                              isems[b]).wait()

    def start_write(i, b):
        pltpu.async_copy(outs[b].at[:, pl.ds(0, _DP)],
                         tp_hbm.at[pl.ds(i * (_W // 2), _W // 2)], osems[b])

    def wait_write(i, b):
        pltpu.make_async_copy(outs[b].at[:, pl.ds(0, _DP)],
                              tp_hbm.at[pl.ds(i * (_W // 2), _W // 2)],
                              osems[b]).wait()

    def transpose(b, nl):
        rvecs = []
        cadds = []
        for lg in range(nl // _L):
            l = lane + lg * _L
            rvecs.append(lax.shift_right_logical(l, 1))
            cadds.append((l & 1) * _D)

        @plsc.parallel_loop(0, _D, unroll=8)
        def _(c):
            for lg in range(nl // _L):
                vals = ins[b][c, pl.ds(lg * _L, _L)]
                plsc.store_scatter(outs[b], [rvecs[lg], cadds[lg] + c], vals)

    nloc = _NCH // _NW + 1
    start_read(wid, 0)

    def chunk_body(g, carry):
        for b in range(2):
            gg = 2 * g + b
            i = wid + gg * _NW

            @pl.when(i < _NCH)
            def _():
                wait_read(i, b)

                @pl.when(i + _NW < _NCH)
                def _():
                    start_read(i + _NW, 1 - b)

                @pl.when(gg >= 2)
                def _():
                    wait_write(i - 2 * _NW, b)

                transpose(b, _W)
                start_write(i, b)
        return carry

    lax.fori_loop(0, (nloc + 1) // 2, chunk_body, 0)
    last = wid + (_NCH // _NW - 1) * _NW
    wait_write(last, (_NCH // _NW - 1) % 2)
    wait_write(last - _NW, (_NCH // _NW) % 2)

    @pl.when(wid == _NW - 1)
    def _():
        pltpu.sync_copy(tail_hbm, ins[0])
        transpose(0, _D)
        pltpu.sync_copy(outs[0].at[pl.ds(0, _D // 2), pl.ds(0, _DP)],
                        tp_hbm.at[pl.ds(_VTAIL // 2, _D // 2)])


def _emb_body(idx_hbm, tablep_hbm, out_hbm, idx_v, lvecs_v, pidx_v, po_v,
              rows0, rows1, rows2, rows3, tr0, tr1,
              gsem0, gsem1, gsem2, gsem3, osem0, osem1):
    rows = (rows0, rows1, rows2, rows3)
    trs = (tr0, tr1)
    gsems = (gsem0, gsem1, gsem2, gsem3)
    osems = (osem0, osem1)
    wid = lax.axis_index("s") * _NC + lax.axis_index("c")
    b0 = wid * _BPW

    lane = lax.iota(jnp.int32, _L)

    def fill_body(i, v):
        lvecs_v[i, :] = v
        return v + 1

    lax.fori_loop(0, _BPW, fill_body, lane * 0)

    def prep_pidx(tl, b):
        for g in range(_BPW // _L):
            sl = pl.ds(g * _L, _L)
            v = idx_v[tl, sl]
            pidx_v[b, sl] = lax.shift_right_logical(v, 1)
            po_v[b, sl] = (v & 1) * _D

    def start_gather(tl, b):
        pltpu.async_copy(tablep_hbm.at[pidx_v.at[b]], rows[b], gsems[b])

    def wait_gather(tl, b):
        pltpu.make_async_copy(tablep_hbm.at[pidx_v.at[b]], rows[b],
                              gsems[b]).wait()

    def start_write(t, b):
        pltpu.async_copy(trs[b].at[:, pl.ds(0, _BPW)],
                         out_hbm.at[t, :, pl.ds(b0, _BPW)], osems[b])

    def wait_write(t, b):
        pltpu.make_async_copy(trs[b].at[:, pl.ds(0, _BPW)],
                              out_hbm.at[t, :, pl.ds(b0, _BPW)],
                              osems[b]).wait()

    def transpose_scale(rb, wb):
        cids = [lane + gc * _L for gc in range(_D // _L)]

        @plsc.parallel_loop(0, _BPW, unroll=8)
        def _(l):
            lvec = lvecs_v[l, :]
            po = po_v[rb, pl.ds(l, _L)][0]
            for gc in range(_D // _L):
                vals = rows[rb][l, pl.ds(po + gc * _L, _L)]
                plsc.store_scatter(trs[wb], [cids[gc], lvec], vals * _SCALE)

    for t0, ht in ((0, 96), (96, 104)):
        pltpu.sync_copy(idx_hbm.at[pl.ds(t0, ht), pl.ds(b0, _BPW)],
                        idx_v.at[pl.ds(0, ht)])
        for tl in range(3):
            prep_pidx(tl, tl)
            start_gather(tl, tl)

        def quad_body(g, carry):
            for b in range(4):
                tl = 4 * g + b
                t = t0 + tl
                wb = b % 2
                wait_gather(tl, b)

                @pl.when(tl + 3 < ht)
                def _():
                    prep_pidx(tl + 3, (b + 3) % 4)
                    start_gather(tl + 3, (b + 3) % 4)

                @pl.when(tl >= 2)
                def _():
                    wait_write(t - 2, wb)

                transpose_scale(b, wb)
                start_write(t, wb)
            return carry

        lax.fori_loop(0, ht // 4, quad_body, 0)
        wait_write(t0 + ht - 2, 0)
        wait_write(t0 + ht - 1, 1)


def _mesh():
    return plsc.VectorSubcoreMesh(
        core_axis_name="c", subcore_axis_name="s",
        num_cores=_NC, num_subcores=_NS,
    )


@jax.jit
def _run(idx_tb, tab_t, tail):
    fmt = pl.kernel(
        _fmt_body,
        out_type=jax.ShapeDtypeStruct((_NPAIR, _DP), jnp.float32),
        mesh=_mesh(),
        scratch_types=[
            pltpu.VMEM((_D, _W), jnp.float32),
            pltpu.VMEM((_D, _W), jnp.float32),
            pltpu.VMEM((_W // 2, _TPAD), jnp.float32),
            pltpu.VMEM((_W // 2, _TPAD), jnp.float32),
            pltpu.SemaphoreType.DMA,
            pltpu.SemaphoreType.DMA,
            pltpu.SemaphoreType.DMA,
            pltpu.SemaphoreType.DMA,
        ],
        compiler_params=pltpu.CompilerParams(
            use_tc_tiling_on_sc=True, needs_layout_passes=False),
    )
    tablep = fmt(tab_t, tail)

    emb = pl.kernel(
        _emb_body,
        out_type=jax.ShapeDtypeStruct((_NT, _D, _NB), jnp.float32),
        mesh=_mesh(),
        scratch_types=[
            pltpu.VMEM((104, _BPW), jnp.int32),
            pltpu.VMEM((_BPW, _L), jnp.int32),
            pltpu.VMEM((4, _BPW), jnp.int32),
            pltpu.VMEM((4, _BPW + _L), jnp.int32),
            pltpu.VMEM((_BPW, _DP), jnp.float32),
            pltpu.VMEM((_BPW, _DP), jnp.float32),
            pltpu.VMEM((_BPW, _DP), jnp.float32),
            pltpu.VMEM((_BPW, _DP), jnp.float32),
            pltpu.VMEM((_D, _TPAD), jnp.float32),
            pltpu.VMEM((_D, _TPAD), jnp.float32),
            pltpu.SemaphoreType.DMA,
            pltpu.SemaphoreType.DMA,
            pltpu.SemaphoreType.DMA,
            pltpu.SemaphoreType.DMA,
            pltpu.SemaphoreType.DMA,
            pltpu.SemaphoreType.DMA,
        ],
        compiler_params=pltpu.CompilerParams(
            use_tc_tiling_on_sc=True, needs_layout_passes=False),
    )
    return emb(idx_tb, tablep)


def kernel(inputs, table):
    idx_tb = inputs.T
    tab_t = table.T
    tail = jnp.pad(lax.slice(tab_t, (0, _VTAIL), (_D, _V)),
                   ((0, 0), (0, _W - (_V - _VTAIL))))
    out_tcb = _run(idx_tb, tab_t, tail)
    return out_tcb.transpose(2, 0, 1)


# speedup vs baseline: 1.3520x; 1.3497x over previous
import functools

import jax
import jax.numpy as jnp
from jax import lax
from jax.experimental import pallas as pl
from jax.experimental.pallas import tpu as pltpu
from jax.experimental.pallas import tpu_sc as plsc

_D = 64
_B = 4096 * 200
_NC, _NS, _L = 2, 16, 16
_NW = _NC * _NS
_BPW = _B // _NW
_CHUNK = 800
_NCHUNKS = _BPW // _CHUNK
_SCALE = 8.0


def _emb_body(idx_hbm, table_hbm, out_hbm, idx_v, rows0, rows1,
              gsem0, gsem1, osem0, osem1):
    rows = (rows0, rows1)
    gsems = (gsem0, gsem1)
    osems = (osem0, osem1)
    wid = lax.axis_index("s") * _NC + lax.axis_index("c")
    base = wid * _BPW
    pltpu.sync_copy(idx_hbm.at[pl.ds(base, _BPW)], idx_v)

    def start_gather(c, b):
        pltpu.async_copy(
            table_hbm.at[idx_v.at[pl.ds(c * _CHUNK, _CHUNK)]], rows[b],
            gsems[b])

    def scale(b):
        def scale_body(i, carry):
            for j in range(_D // _L):
                sl = pl.ds(j * _L, _L)
                rows[b][i, sl] = rows[b][i, sl] * _SCALE
            return carry
        lax.fori_loop(0, _CHUNK, scale_body, 0, unroll=4)

    start_gather(0, 0)

    def pair_body(g, carry):
        for b in range(2):
            c = 2 * g + b
            nb = 1 - b
            pltpu.make_async_copy(
                table_hbm.at[idx_v.at[pl.ds(c * _CHUNK, _CHUNK)]], rows[b],
                gsems[b]).wait()
            @pl.when(c >= 1)
            def _():
                pltpu.make_async_copy(
                    rows[nb], out_hbm.at[pl.ds(base, _CHUNK)],
                    osems[nb]).wait()

            @pl.when(c + 1 < _NCHUNKS)
            def _():
                start_gather(c + 1, nb)

            scale(b)
            pltpu.async_copy(
                rows[b], out_hbm.at[pl.ds(base + c * _CHUNK, _CHUNK)],
                osems[b])
        return carry

    lax.fori_loop(0, _NCHUNKS // 2, pair_body, 0)
    lb = (_NCHUNKS - 1) % 2
    pltpu.make_async_copy(
        rows[lb], out_hbm.at[pl.ds(base, _CHUNK)], osems[lb]).wait()


@jax.jit
def _emb(idx_flat, table):
    mesh = plsc.VectorSubcoreMesh(
        core_axis_name="c", subcore_axis_name="s",
        num_cores=_NC, num_subcores=_NS,
    )
    f = pl.kernel(
        _emb_body,
        out_type=jax.ShapeDtypeStruct((_B, _D), jnp.float32),
        mesh=mesh,
        scratch_types=[
            pltpu.VMEM((_BPW,), jnp.int32),
            pltpu.VMEM((_CHUNK, _D), jnp.float32),
            pltpu.VMEM((_CHUNK, _D), jnp.float32),
            pltpu.SemaphoreType.DMA,
            pltpu.SemaphoreType.DMA,
            pltpu.SemaphoreType.DMA,
            pltpu.SemaphoreType.DMA,
        ],
        compiler_params=pltpu.CompilerParams(use_tc_tiling_on_sc=False),
    )
    return f(idx_flat, table)


def kernel(inputs, table):
    flat = inputs.reshape(-1)
    out = _emb(flat, table)
    return out.reshape(inputs.shape + (_D,))
